# Initial kernel scaffold; baseline (speedup 1.0000x reference)
import jax
import jax.numpy as jnp
from jax.experimental import pallas as pl


def kernel(input_x, edge_index, W, b):
    raise NotImplementedError("write your pallas kernel here")


# speedup vs baseline: 27.1667x; 27.1667x over previous
import functools
import jax
import jax.numpy as jnp
from jax import lax
from jax.experimental import pallas as pl
from jax.experimental.pallas import tpu as pltpu
from jax.experimental.pallas import tpu_sc as plsc

N_NODES = 10000
N_EDGES = 320000
D = 128

NC, NS = 2, 16
NW = NC * NS
NPAD = 10240
EPW = 10240
EPAD = NW * EPW
BATCH = 256
NSTEP = EPW // BATCH
IPR = 128
KPB = BATCH // IPR
ROWS_T = NPAD // NS

def _deg_body(dst_hbm, out_hbm, idx_v, ones_v, zb_v, acc_sh):
    c = lax.axis_index("c")
    s = lax.axis_index("s")
    wid = c * NS + s

    for j in range(IPR // 16):
        ones_v[pl.ds(16 * j, 16)] = jnp.ones((16,), jnp.float32)
    for j in range(ROWS_T // 16):
        zb_v[pl.ds(16 * j, 16)] = jnp.zeros((16,), jnp.float32)
    pltpu.sync_copy(zb_v, acc_sh.at[pl.ds(s * ROWS_T, ROWS_T)])
    plsc.subcore_barrier()

    def step(i, carry):
        base = wid * (EPW // IPR) + i * KPB
        pltpu.sync_copy(dst_hbm.at[pl.ds(base, KPB)], idx_v)
        for j in range(KPB):
            pltpu.sync_copy(ones_v, acc_sh.at[idx_v.at[j]], add=True)
        return carry

    lax.fori_loop(0, NSTEP, step, 0)
    plsc.subcore_barrier()
    pltpu.sync_copy(
        acc_sh.at[pl.ds(s * ROWS_T, ROWS_T)],
        out_hbm.at[c, pl.ds(s * ROWS_T, ROWS_T)],
    )


def _seg_body(src_hbm, dst_hbm, g_hbm, out_hbm, src_v, dst_v, rows_v, acc_sh, sem):
    c = lax.axis_index("c")
    s = lax.axis_index("s")
    wid = c * NS + s

    def zfill(r, carry):
        for j in range(D // 16):
            rows_v[r, pl.ds(16 * j, 16)] = jnp.zeros((16,), jnp.float32)
        return carry

    lax.fori_loop(0, BATCH, zfill, 0)
    for t in range(ROWS_T // BATCH):
        pltpu.sync_copy(rows_v, acc_sh.at[pl.ds(s * ROWS_T + t * BATCH, BATCH)])
    rem = ROWS_T % BATCH
    if rem:
        pltpu.sync_copy(
            rows_v.at[pl.ds(0, rem)],
            acc_sh.at[pl.ds(s * ROWS_T + (ROWS_T // BATCH) * BATCH, rem)],
        )
    plsc.subcore_barrier()

    def step(i, carry):
        base = wid * (EPW // IPR) + i * KPB
        pltpu.sync_copy(src_hbm.at[pl.ds(base, KPB)], src_v)
        pltpu.sync_copy(dst_hbm.at[pl.ds(base, KPB)], dst_v)
        for j in range(KPB):
            pltpu.async_copy(g_hbm.at[src_v.at[j]], rows_v.at[pl.ds(j * IPR, IPR)], sem)
        for j in range(KPB):
            pltpu.make_async_copy(
                g_hbm.at[src_v.at[j]], rows_v.at[pl.ds(j * IPR, IPR)], sem
            ).wait()
        for j in range(KPB):
            pltpu.sync_copy(
                rows_v.at[pl.ds(j * IPR, IPR)], acc_sh.at[dst_v.at[j]], add=True
            )
        return carry

    lax.fori_loop(0, NSTEP, step, 0)
    plsc.subcore_barrier()
    pltpu.sync_copy(
        acc_sh.at[pl.ds(s * ROWS_T, ROWS_T)],
        out_hbm.at[c, pl.ds(s * ROWS_T, ROWS_T)],
    )


def _lin_body(x_ref, w_ref, degp_ref, h_ref, g_ref):
    deg = degp_ref[0] + degp_ref[1] + 1.0
    dis = lax.rsqrt(deg)
    h = jnp.dot(x_ref[...], w_ref[...], preferred_element_type=jnp.float32)
    h_ref[...] = h
    g_ref[...] = dis * h


def _comb_body(sp_ref, h_ref, degp_ref, b_ref, out_ref):
    deg = degp_ref[0] + degp_ref[1] + 1.0
    dis = lax.rsqrt(deg)
    s_sum = sp_ref[0] + sp_ref[1]
    out_ref[...] = dis * s_sum + (dis * dis) * h_ref[...] + b_ref[...]


_BLK = 1024
_GRID = NPAD // _BLK


@functools.lru_cache(maxsize=None)
def _sc_kernels():
    mesh = plsc.VectorSubcoreMesh(
        core_axis_name="c", subcore_axis_name="s", num_cores=NC, num_subcores=NS
    )
    deg_k = pl.kernel(
        _deg_body,
        out_type=jax.ShapeDtypeStruct((NC, NPAD), jnp.float32),
        mesh=mesh,
        scratch_types=[
            pltpu.VMEM((KPB, IPR), jnp.int32),
            pltpu.VMEM((IPR,), jnp.float32),
            pltpu.VMEM((ROWS_T,), jnp.float32),
            pltpu.VMEM_SHARED((NPAD,), jnp.float32),
        ],
    )
    seg_k = pl.kernel(
        _seg_body,
        out_type=jax.ShapeDtypeStruct((NC, NPAD, D), jnp.float32),
        mesh=mesh,
        scratch_types=[
            pltpu.VMEM((KPB, IPR), jnp.int32),
            pltpu.VMEM((KPB, IPR), jnp.int32),
            pltpu.VMEM((BATCH, D), jnp.float32),
            pltpu.VMEM_SHARED((NPAD, D), jnp.float32),
            pltpu.SemaphoreType.DMA,
        ],
    )
    return deg_k, seg_k


def kernel(input_x, edge_index, W, b):
    x = input_x.astype(jnp.float32)
    src = edge_index[0].astype(jnp.int32)
    dst = edge_index[1].astype(jnp.int32)

    n_extra = EPAD - N_EDGES
    fill = N_NODES + (jnp.arange(n_extra, dtype=jnp.int32) % (NPAD - N_NODES))
    src_p = jnp.concatenate([src, fill]).reshape(EPAD // IPR, IPR)
    dst_p = jnp.concatenate([dst, fill]).reshape(EPAD // IPR, IPR)

    x_pad = jnp.pad(x, ((0, NPAD - N_NODES), (0, 0)))

    deg_kernel, seg_kernel = _sc_kernels()
    degp = deg_kernel(dst_p)
    degp_col = degp.reshape(NC, NPAD, 1)

    h, g = pl.pallas_call(
        _lin_body,
        grid=(_GRID,),
        in_specs=[
            pl.BlockSpec((_BLK, D), lambda i: (i, 0)),
            pl.BlockSpec((D, D), lambda i: (0, 0)),
            pl.BlockSpec((NC, _BLK, 1), lambda i: (0, i, 0)),
        ],
        out_specs=[
            pl.BlockSpec((_BLK, D), lambda i: (i, 0)),
            pl.BlockSpec((_BLK, D), lambda i: (i, 0)),
        ],
        out_shape=[
            jax.ShapeDtypeStruct((NPAD, D), jnp.float32),
            jax.ShapeDtypeStruct((NPAD, D), jnp.float32),
        ],
    )(x_pad, W.astype(jnp.float32), degp_col)

    sp = seg_kernel(src_p, dst_p, g)

    out_pad = pl.pallas_call(
        _comb_body,
        grid=(_GRID,),
        in_specs=[
            pl.BlockSpec((NC, _BLK, D), lambda i: (0, i, 0)),
            pl.BlockSpec((_BLK, D), lambda i: (i, 0)),
            pl.BlockSpec((NC, _BLK, 1), lambda i: (0, i, 0)),
            pl.BlockSpec((1, D), lambda i: (0, 0)),
        ],
        out_specs=pl.BlockSpec((_BLK, D), lambda i: (i, 0)),
        out_shape=jax.ShapeDtypeStruct((NPAD, D), jnp.float32),
    )(sp, h, degp_col, b.astype(jnp.float32).reshape(1, D))

    return out_pad[:N_NODES]


# speedup vs baseline: 41.1851x; 1.5160x over previous
import functools
import jax
import jax.numpy as jnp
from jax import lax
from jax.experimental import pallas as pl
from jax.experimental.pallas import tpu as pltpu
from jax.experimental.pallas import tpu_sc as plsc

N_NODES = 10000
N_EDGES = 320000
D = 128

NC, NS = 2, 16
NW = NC * NS
NPAD = 10240
EPW = 10240
EPAD = NW * EPW
IPR = 128
NT = EPW // IPR
ICH = 40
ROWS_T = NPAD // NS

def _deg_body(dst_hbm, out_hbm, idx_v, ones_v, zb_v, acc_sh):
    c = lax.axis_index("c")
    s = lax.axis_index("s")
    wid = c * NS + s

    for j in range(IPR // 16):
        ones_v[pl.ds(16 * j, 16)] = jnp.ones((16,), jnp.float32)
    for j in range(ROWS_T // 16):
        zb_v[pl.ds(16 * j, 16)] = jnp.zeros((16,), jnp.float32)
    pltpu.sync_copy(zb_v, acc_sh.at[pl.ds(s * ROWS_T, ROWS_T)])
    pltpu.sync_copy(dst_hbm.at[pl.ds(wid * NT, NT)], idx_v)
    plsc.subcore_barrier()

    def step(i, carry):
        pltpu.sync_copy(ones_v, acc_sh.at[idx_v.at[i]], add=True)
        return carry

    lax.fori_loop(0, NT, step, 0)
    plsc.subcore_barrier()
    pltpu.sync_copy(
        acc_sh.at[pl.ds(s * ROWS_T, ROWS_T)],
        out_hbm.at[c, pl.ds(s * ROWS_T, ROWS_T)],
    )


def _seg_body(src_hbm, dst_hbm, g_hbm, out_hbm, src_v, dst_v, buf0, buf1, acc_sh, sem0, sem1):
    c = lax.axis_index("c")
    s = lax.axis_index("s")
    wid = c * NS + s
    bufs = (buf0, buf1)
    sems = (sem0, sem1)

    def zfill(r, carry):
        for j in range(D // 16):
            buf0[r, pl.ds(16 * j, 16)] = jnp.zeros((16,), jnp.float32)
        return carry

    lax.fori_loop(0, IPR, zfill, 0)
    for t in range(ROWS_T // IPR):
        pltpu.sync_copy(buf0, acc_sh.at[pl.ds(s * ROWS_T + t * IPR, IPR)])
    plsc.subcore_barrier()

    for half in range(NT // ICH):
        hbase = wid * NT + half * ICH
        pltpu.sync_copy(src_hbm.at[pl.ds(hbase, ICH)], src_v)
        pltpu.sync_copy(dst_hbm.at[pl.ds(hbase, ICH)], dst_v)
        pltpu.async_copy(g_hbm.at[src_v.at[0]], buf0, sem0)

        def pair(p, carry):
            for b in range(2):
                i = p * 2 + b
                nxt = i + 1

                @pl.when(nxt < ICH)
                def _():
                    pltpu.async_copy(
                        g_hbm.at[src_v.at[nxt]], bufs[1 - b], sems[1 - b]
                    )

                pltpu.make_async_copy(
                    g_hbm.at[src_v.at[i]], bufs[b], sems[b]
                ).wait()
                pltpu.sync_copy(bufs[b], acc_sh.at[dst_v.at[i]], add=True)
            return carry

        lax.fori_loop(0, ICH // 2, pair, 0)

    plsc.subcore_barrier()
    pltpu.sync_copy(
        acc_sh.at[pl.ds(s * ROWS_T, ROWS_T)],
        out_hbm.at[c, pl.ds(s * ROWS_T, ROWS_T)],
    )


def _mm_body(x_ref, w_ref, h_ref):
    h_ref[...] = jnp.dot(x_ref[...], w_ref[...], preferred_element_type=jnp.float32)


def _scale_body(h_ref, degp_ref, g_ref):
    deg = degp_ref[0] + degp_ref[1] + 1.0
    g_ref[...] = lax.rsqrt(deg) * h_ref[...]


def _comb_body(sp_ref, h_ref, degp_ref, b_ref, out_ref):
    deg = degp_ref[0] + degp_ref[1] + 1.0
    dis = lax.rsqrt(deg)
    s_sum = sp_ref[0] + sp_ref[1]
    out_ref[...] = dis * s_sum + (dis * dis) * h_ref[...] + b_ref[...]


_BLK = 1024
_GRID = NPAD // _BLK


@functools.lru_cache(maxsize=None)
def _sc_kernels():
    mesh = plsc.VectorSubcoreMesh(
        core_axis_name="c", subcore_axis_name="s", num_cores=NC, num_subcores=NS
    )
    deg_k = pl.kernel(
        _deg_body,
        out_type=jax.ShapeDtypeStruct((NC, NPAD), jnp.float32),
        mesh=mesh,
        scratch_types=[
            pltpu.VMEM((NT, IPR), jnp.int32),
            pltpu.VMEM((IPR,), jnp.float32),
            pltpu.VMEM((ROWS_T,), jnp.float32),
            pltpu.VMEM_SHARED((NPAD,), jnp.float32),
        ],
    )
    seg_k = pl.kernel(
        _seg_body,
        out_type=jax.ShapeDtypeStruct((NC, NPAD, D), jnp.float32),
        mesh=mesh,
        scratch_types=[
            pltpu.VMEM((ICH, IPR), jnp.int32),
            pltpu.VMEM((ICH, IPR), jnp.int32),
            pltpu.VMEM((IPR, D), jnp.float32),
            pltpu.VMEM((IPR, D), jnp.float32),
            pltpu.VMEM_SHARED((NPAD, D), jnp.float32),
            pltpu.SemaphoreType.DMA,
            pltpu.SemaphoreType.DMA,
        ],
    )
    return deg_k, seg_k


def kernel(input_x, edge_index, W, b):
    x = input_x.astype(jnp.float32)
    src = edge_index[0].astype(jnp.int32)
    dst = edge_index[1].astype(jnp.int32)

    n_extra = EPAD - N_EDGES
    fill = N_NODES + (jnp.arange(n_extra, dtype=jnp.int32) % (NPAD - N_NODES))
    src_p = jnp.concatenate([src, fill]).reshape(EPAD // IPR, IPR)
    dst_p = jnp.concatenate([dst, fill]).reshape(EPAD // IPR, IPR)

    x_pad = jnp.pad(x, ((0, NPAD - N_NODES), (0, 0)))

    deg_kernel, seg_kernel = _sc_kernels()
    degp = deg_kernel(dst_p)
    degp_col = degp.reshape(NC, NPAD, 1)

    h = pl.pallas_call(
        _mm_body,
        grid=(_GRID,),
        in_specs=[
            pl.BlockSpec((_BLK, D), lambda i: (i, 0)),
            pl.BlockSpec((D, D), lambda i: (0, 0)),
        ],
        out_specs=pl.BlockSpec((_BLK, D), lambda i: (i, 0)),
        out_shape=jax.ShapeDtypeStruct((NPAD, D), jnp.float32),
    )(x_pad, W.astype(jnp.float32))

    g = pl.pallas_call(
        _scale_body,
        grid=(_GRID,),
        in_specs=[
            pl.BlockSpec((_BLK, D), lambda i: (i, 0)),
            pl.BlockSpec((NC, _BLK, 1), lambda i: (0, i, 0)),
        ],
        out_specs=pl.BlockSpec((_BLK, D), lambda i: (i, 0)),
        out_shape=jax.ShapeDtypeStruct((NPAD, D), jnp.float32),
    )(h, degp_col)

    sp = seg_kernel(src_p, dst_p, g)

    out_pad = pl.pallas_call(
        _comb_body,
        grid=(_GRID,),
        in_specs=[
            pl.BlockSpec((NC, _BLK, D), lambda i: (0, i, 0)),
            pl.BlockSpec((_BLK, D), lambda i: (i, 0)),
            pl.BlockSpec((NC, _BLK, 1), lambda i: (0, i, 0)),
            pl.BlockSpec((1, D), lambda i: (0, 0)),
        ],
        out_specs=pl.BlockSpec((_BLK, D), lambda i: (i, 0)),
        out_shape=jax.ShapeDtypeStruct((NPAD, D), jnp.float32),
    )(sp, h, degp_col, b.astype(jnp.float32).reshape(1, D))

    return out_pad[:N_NODES]


# speedup vs baseline: 41.4365x; 1.0061x over previous
import functools
import jax
import jax.numpy as jnp
from jax import lax
from jax.experimental import pallas as pl
from jax.experimental.pallas import tpu as pltpu
from jax.experimental.pallas import tpu_sc as plsc

N_NODES = 10000
N_EDGES = 320000
D = 128

NC, NS = 2, 16
NW = NC * NS
NPAD = 10240
EPW = 10240
EPAD = NW * EPW
IPR = 128
NT = EPW // IPR
ROWS_T = NPAD // NS
SPR = 128
NT2 = EPW // SPR
SCH = NT2 // 2
NBUF = 2

def _deg_body(dst_hbm, out_hbm, idx_v, ones_v, zb_v, acc_sh, sem):
    c = lax.axis_index("c")
    s = lax.axis_index("s")
    wid = c * NS + s

    for j in range(IPR // 16):
        ones_v[pl.ds(16 * j, 16)] = jnp.ones((16,), jnp.float32)
    for j in range(ROWS_T // 16):
        zb_v[pl.ds(16 * j, 16)] = jnp.zeros((16,), jnp.float32)
    pltpu.sync_copy(zb_v, acc_sh.at[pl.ds(s * ROWS_T, ROWS_T)])
    pltpu.sync_copy(dst_hbm.at[pl.ds(wid * NT, NT)], idx_v)
    plsc.subcore_barrier()

    def step(q, carry):
        for j in range(8):
            pltpu.async_copy(ones_v, acc_sh.at[idx_v.at[q * 8 + j]], sem, add=True)
        for j in range(8):
            pltpu.make_async_copy(ones_v, acc_sh.at[idx_v.at[q * 8 + j]], sem).wait()
        return carry

    lax.fori_loop(0, NT // 8, step, 0)
    plsc.subcore_barrier()
    pltpu.sync_copy(
        acc_sh.at[pl.ds(s * ROWS_T, ROWS_T)],
        out_hbm.at[c, pl.ds(s * ROWS_T, ROWS_T)],
    )


def _seg_body(src_hbm, dst_hbm, g_hbm, out_hbm, src_v, dst_v,
              buf0, buf1, acc_sh, gs0, gs1):
    c = lax.axis_index("c")
    s = lax.axis_index("s")
    wid = c * NS + s
    bufs = (buf0, buf1)
    gsem = (gs0, gs1)

    def zfill(r, carry):
        for j in range(D // 16):
            buf0[r, pl.ds(16 * j, 16)] = jnp.zeros((16,), jnp.float32)
        return carry

    lax.fori_loop(0, SPR, zfill, 0)
    for t in range(ROWS_T // SPR):
        pltpu.sync_copy(buf0, acc_sh.at[pl.ds(s * ROWS_T + t * SPR, SPR)])
    plsc.subcore_barrier()

    def gather(i, b):
        pltpu.async_copy(g_hbm.at[src_v.at[i]], bufs[b], gsem[b])

    def gather_wait(i, b):
        pltpu.make_async_copy(g_hbm.at[src_v.at[i]], bufs[b], gsem[b]).wait()

    for half in range(NT2 // SCH):
        hbase = wid * NT2 + half * SCH
        pltpu.sync_copy(src_hbm.at[pl.ds(hbase, SCH)], src_v)
        pltpu.sync_copy(dst_hbm.at[pl.ds(hbase, SCH)], dst_v)
        gather(0, 0)

        def pair(p, carry):
            for b in range(NBUF):
                i = p * NBUF + b
                nxt = i + 1

                @pl.when(nxt < SCH)
                def _():
                    gather(nxt, 1 - b)

                gather_wait(i, b)
                pltpu.sync_copy(bufs[b], acc_sh.at[dst_v.at[i]], add=True)
            return carry

        lax.fori_loop(0, SCH // NBUF, pair, 0)

    plsc.subcore_barrier()
    pltpu.sync_copy(
        acc_sh.at[pl.ds(s * ROWS_T, ROWS_T)],
        out_hbm.at[c, pl.ds(s * ROWS_T, ROWS_T)],
    )


def _mm_body(x_ref, w_ref, h_ref):
    h_ref[...] = jnp.dot(x_ref[...], w_ref[...],
                         preferred_element_type=jnp.float32)


def _scale_body(h_ref, degp_ref, g_ref):
    deg = degp_ref[0] + degp_ref[1] + 1.0
    g_ref[...] = lax.rsqrt(deg) * h_ref[...]


def _comb_body(sp_ref, h_ref, degp_ref, b_ref, out_ref):
    deg = degp_ref[0] + degp_ref[1] + 1.0
    dis = lax.rsqrt(deg)
    s_sum = sp_ref[0] + sp_ref[1]
    out_ref[...] = dis * s_sum + (dis * dis) * h_ref[...] + b_ref[...]


_BLK = 1024
_GRID = NPAD // _BLK


@functools.lru_cache(maxsize=None)
def _sc_kernels():
    mesh = plsc.VectorSubcoreMesh(
        core_axis_name="c", subcore_axis_name="s", num_cores=NC, num_subcores=NS
    )
    deg_k = pl.kernel(
        _deg_body,
        out_type=jax.ShapeDtypeStruct((NC, NPAD), jnp.float32),
        mesh=mesh,
        scratch_types=[
            pltpu.VMEM((NT, IPR), jnp.int32),
            pltpu.VMEM((IPR,), jnp.float32),
            pltpu.VMEM((ROWS_T,), jnp.float32),
            pltpu.VMEM_SHARED((NPAD,), jnp.float32),
            pltpu.SemaphoreType.DMA,
        ],
    )
    seg_k = pl.kernel(
        _seg_body,
        out_type=jax.ShapeDtypeStruct((NC, NPAD, D), jnp.float32),
        mesh=mesh,
        scratch_types=(
            [
                pltpu.VMEM((SCH, SPR), jnp.int32),
                pltpu.VMEM((SCH, SPR), jnp.int32),
            ]
            + [pltpu.VMEM((SPR, D), jnp.float32)] * NBUF
            + [pltpu.VMEM_SHARED((NPAD, D), jnp.float32)]
            + [pltpu.SemaphoreType.DMA] * NBUF
        ),
    )
    return deg_k, seg_k


def kernel(input_x, edge_index, W, b):
    x = input_x.astype(jnp.float32)
    src = edge_index[0].astype(jnp.int32)
    dst = edge_index[1].astype(jnp.int32)

    n_extra = EPAD - N_EDGES
    fill = N_NODES + (jnp.arange(n_extra, dtype=jnp.int32) % (NPAD - N_NODES))
    src_flat = jnp.concatenate([src, fill])
    dst_flat = jnp.concatenate([dst, fill])
    src_p = src_flat.reshape(EPAD // SPR, SPR)
    dst_p = dst_flat.reshape(EPAD // SPR, SPR)
    dst_p128 = dst_flat.reshape(EPAD // IPR, IPR)

    x_pad = jnp.pad(x, ((0, NPAD - N_NODES), (0, 0)))

    deg_kernel, seg_kernel = _sc_kernels()
    degp = deg_kernel(dst_p128)
    degp_col = degp.reshape(NC, NPAD, 1)

    h = pl.pallas_call(
        _mm_body,
        grid=(_GRID,),
        in_specs=[
            pl.BlockSpec((_BLK, D), lambda i: (i, 0)),
            pl.BlockSpec((D, D), lambda i: (0, 0)),
        ],
        out_specs=pl.BlockSpec((_BLK, D), lambda i: (i, 0)),
        out_shape=jax.ShapeDtypeStruct((NPAD, D), jnp.float32),
    )(x_pad, W.astype(jnp.float32))

    g = pl.pallas_call(
        _scale_body,
        grid=(_GRID,),
        in_specs=[
            pl.BlockSpec((_BLK, D), lambda i: (i, 0)),
            pl.BlockSpec((NC, _BLK, 1), lambda i: (0, i, 0)),
        ],
        out_specs=pl.BlockSpec((_BLK, D), lambda i: (i, 0)),
        out_shape=jax.ShapeDtypeStruct((NPAD, D), jnp.float32),
    )(h, degp_col)

    sp = seg_kernel(src_p, dst_p, g)

    out_pad = pl.pallas_call(
        _comb_body,
        grid=(_GRID,),
        in_specs=[
            pl.BlockSpec((NC, _BLK, D), lambda i: (0, i, 0)),
            pl.BlockSpec((_BLK, D), lambda i: (i, 0)),
            pl.BlockSpec((NC, _BLK, 1), lambda i: (0, i, 0)),
            pl.BlockSpec((1, D), lambda i: (0, 0)),
        ],
        out_specs=pl.BlockSpec((_BLK, D), lambda i: (i, 0)),
        out_shape=jax.ShapeDtypeStruct((NPAD, D), jnp.float32),
    )(sp, h, degp_col, b.astype(jnp.float32).reshape(1, D))

    return out_pad[:N_NODES]


# speedup vs baseline: 42.0010x; 1.0136x over previous
import functools
import jax
import jax.numpy as jnp
from jax import lax
from jax.experimental import pallas as pl
from jax.experimental.pallas import tpu as pltpu
from jax.experimental.pallas import tpu_sc as plsc

N_NODES = 10000
N_EDGES = 320000
D = 128

NC, NS = 2, 16
NW = NC * NS
NPAD = 10240
EPW = 10240
EPAD = NW * EPW
IPR = 128
NT = EPW // IPR
ROWS_T = NPAD // NS
SPR = 128
NT2 = EPW // SPR
SCH = NT2 // 2
NBUF = 2

def _deg_body(dst_hbm, out_hbm, idx_v, ones_v, zb_v, acc_sh, sem):
    c = lax.axis_index("c")
    s = lax.axis_index("s")
    wid = c * NS + s

    for j in range(IPR // 16):
        ones_v[pl.ds(16 * j, 16)] = jnp.ones((16,), jnp.float32)
    for j in range(ROWS_T // 16):
        zb_v[pl.ds(16 * j, 16)] = jnp.zeros((16,), jnp.float32)
    pltpu.sync_copy(zb_v, acc_sh.at[pl.ds(s * ROWS_T, ROWS_T)])
    pltpu.sync_copy(dst_hbm.at[pl.ds(wid * NT, NT)], idx_v)
    plsc.subcore_barrier()

    def step(q, carry):
        for j in range(8):
            pltpu.async_copy(ones_v, acc_sh.at[idx_v.at[q * 8 + j]], sem, add=True)
        for j in range(8):
            pltpu.make_async_copy(ones_v, acc_sh.at[idx_v.at[q * 8 + j]], sem).wait()
        return carry

    lax.fori_loop(0, NT // 8, step, 0)
    plsc.subcore_barrier()
    pltpu.sync_copy(
        acc_sh.at[pl.ds(s * ROWS_T, ROWS_T)],
        out_hbm.at[c, pl.ds(s * ROWS_T, ROWS_T)],
    )


def _seg_body(src_hbm, dst_hbm, g_hbm, out_hbm, src_v, dst_v,
              buf0, buf1, acc_sh, gs0, gs1):
    c = lax.axis_index("c")
    s = lax.axis_index("s")
    wid = c * NS + s
    bufs = (buf0, buf1)
    gsem = (gs0, gs1)

    def zfill(r, carry):
        for j in range(D // 16):
            buf0[r, pl.ds(16 * j, 16)] = jnp.zeros((16,), jnp.float32)
        return carry

    lax.fori_loop(0, SPR, zfill, 0)
    for t in range(ROWS_T // SPR):
        pltpu.sync_copy(buf0, acc_sh.at[pl.ds(s * ROWS_T + t * SPR, SPR)])
    plsc.subcore_barrier()

    def gather(i, b):
        pltpu.async_copy(g_hbm.at[src_v.at[i]], bufs[b], gsem[b])

    def gather_wait(i, b):
        pltpu.make_async_copy(g_hbm.at[src_v.at[i]], bufs[b], gsem[b]).wait()

    for half in range(NT2 // SCH):
        hbase = wid * NT2 + half * SCH
        pltpu.sync_copy(src_hbm.at[pl.ds(hbase, SCH)], src_v)
        pltpu.sync_copy(dst_hbm.at[pl.ds(hbase, SCH)], dst_v)
        gather(0, 0)

        def pair(p, carry):
            for b in range(NBUF):
                i = p * NBUF + b
                nxt = i + 1

                @pl.when(nxt < SCH)
                def _():
                    gather(nxt, 1 - b)

                gather_wait(i, b)
                pltpu.sync_copy(bufs[b], acc_sh.at[dst_v.at[i]], add=True)
            return carry

        lax.fori_loop(0, SCH // NBUF, pair, 0)

    plsc.subcore_barrier()
    pltpu.sync_copy(
        acc_sh.at[pl.ds(s * ROWS_T, ROWS_T)],
        out_hbm.at[c, pl.ds(s * ROWS_T, ROWS_T)],
    )


def _scale_body(x_ref, degp_ref, q_ref):
    deg = degp_ref[0] + degp_ref[1] + 1.0
    q_ref[...] = lax.rsqrt(deg) * x_ref[...]


def _comb_body(sp_ref, x_ref, degp_ref, w_ref, b_ref, out_ref):
    deg = degp_ref[0] + degp_ref[1] + 1.0
    dis = lax.rsqrt(deg)
    t = dis * (sp_ref[0] + sp_ref[1]) + (dis * dis) * x_ref[...]
    out_ref[...] = (
        jnp.dot(t, w_ref[...], preferred_element_type=jnp.float32,
                precision=lax.Precision.HIGHEST)
        + b_ref[...]
    )


_BLK = 1024
_GRID = NPAD // _BLK


@functools.lru_cache(maxsize=None)
def _sc_kernels():
    mesh = plsc.VectorSubcoreMesh(
        core_axis_name="c", subcore_axis_name="s", num_cores=NC, num_subcores=NS
    )
    deg_k = pl.kernel(
        _deg_body,
        out_type=jax.ShapeDtypeStruct((NC, NPAD), jnp.float32),
        mesh=mesh,
        scratch_types=[
            pltpu.VMEM((NT, IPR), jnp.int32),
            pltpu.VMEM((IPR,), jnp.float32),
            pltpu.VMEM((ROWS_T,), jnp.float32),
            pltpu.VMEM_SHARED((NPAD,), jnp.float32),
            pltpu.SemaphoreType.DMA,
        ],
    )
    seg_k = pl.kernel(
        _seg_body,
        out_type=jax.ShapeDtypeStruct((NC, NPAD, D), jnp.float32),
        mesh=mesh,
        scratch_types=(
            [
                pltpu.VMEM((SCH, SPR), jnp.int32),
                pltpu.VMEM((SCH, SPR), jnp.int32),
            ]
            + [pltpu.VMEM((SPR, D), jnp.float32)] * NBUF
            + [pltpu.VMEM_SHARED((NPAD, D), jnp.float32)]
            + [pltpu.SemaphoreType.DMA] * NBUF
        ),
    )
    return deg_k, seg_k


def kernel(input_x, edge_index, W, b):
    x = input_x.astype(jnp.float32)
    src = edge_index[0].astype(jnp.int32)
    dst = edge_index[1].astype(jnp.int32)

    n_extra = EPAD - N_EDGES
    fill = N_NODES + (jnp.arange(n_extra, dtype=jnp.int32) % (NPAD - N_NODES))
    src_flat = jnp.concatenate([src, fill])
    dst_flat = jnp.concatenate([dst, fill])
    src_p = src_flat.reshape(EPAD // SPR, SPR)
    dst_p = dst_flat.reshape(EPAD // SPR, SPR)
    dst_p128 = dst_flat.reshape(EPAD // IPR, IPR)

    x_pad = jnp.pad(x, ((0, NPAD - N_NODES), (0, 0)))

    deg_kernel, seg_kernel = _sc_kernels()
    degp = deg_kernel(dst_p128)
    degp_col = degp.reshape(NC, NPAD, 1)

    q = pl.pallas_call(
        _scale_body,
        grid=(_GRID,),
        in_specs=[
            pl.BlockSpec((_BLK, D), lambda i: (i, 0)),
            pl.BlockSpec((NC, _BLK, 1), lambda i: (0, i, 0)),
        ],
        out_specs=pl.BlockSpec((_BLK, D), lambda i: (i, 0)),
        out_shape=jax.ShapeDtypeStruct((NPAD, D), jnp.float32),
    )(x_pad, degp_col)

    sp = seg_kernel(src_p, dst_p, q)

    out_pad = pl.pallas_call(
        _comb_body,
        grid=(_GRID,),
        in_specs=[
            pl.BlockSpec((NC, _BLK, D), lambda i: (0, i, 0)),
            pl.BlockSpec((_BLK, D), lambda i: (i, 0)),
            pl.BlockSpec((NC, _BLK, 1), lambda i: (0, i, 0)),
            pl.BlockSpec((D, D), lambda i: (0, 0)),
            pl.BlockSpec((1, D), lambda i: (0, 0)),
        ],
        out_specs=pl.BlockSpec((_BLK, D), lambda i: (i, 0)),
        out_shape=jax.ShapeDtypeStruct((NPAD, D), jnp.float32),
    )(sp, x_pad, degp_col, W.astype(jnp.float32),
      b.astype(jnp.float32).reshape(1, D))

    return out_pad[:N_NODES]


# speedup vs baseline: 43.0191x; 1.0242x over previous
import functools
import jax
import jax.numpy as jnp
from jax import lax
from jax.experimental import pallas as pl
from jax.experimental.pallas import tpu as pltpu
from jax.experimental.pallas import tpu_sc as plsc

N_NODES = 10000
N_EDGES = 320000
D = 128

NC, NS = 2, 16
NW = NC * NS
NPAD = 10240
EPW = 10240
EPAD = NW * EPW
IPR = 128
NT = EPW // IPR
ROWS_T = NPAD // NS
SPR = 112
NT2 = 90
SCH = 18
NBUF = 3
EPW2 = NT2 * SPR
EPAD2 = NW * EPW2
REM_Z = ROWS_T - (ROWS_T // SPR) * SPR

def _deg_body(dst_hbm, out_hbm, idx_v, ones_v, zb_v, acc_sh, sem):
    c = lax.axis_index("c")
    s = lax.axis_index("s")
    wid = c * NS + s

    for j in range(IPR // 16):
        ones_v[pl.ds(16 * j, 16)] = jnp.ones((16,), jnp.float32)
    for j in range(ROWS_T // 16):
        zb_v[pl.ds(16 * j, 16)] = jnp.zeros((16,), jnp.float32)
    pltpu.sync_copy(zb_v, acc_sh.at[pl.ds(s * ROWS_T, ROWS_T)])
    pltpu.sync_copy(dst_hbm.at[pl.ds(wid * NT, NT)], idx_v)
    plsc.subcore_barrier()

    def step(q, carry):
        for j in range(8):
            pltpu.async_copy(ones_v, acc_sh.at[idx_v.at[q * 8 + j]], sem, add=True)
        for j in range(8):
            pltpu.make_async_copy(ones_v, acc_sh.at[idx_v.at[q * 8 + j]], sem).wait()
        return carry

    lax.fori_loop(0, NT // 8, step, 0)
    plsc.subcore_barrier()
    pltpu.sync_copy(
        acc_sh.at[pl.ds(s * ROWS_T, ROWS_T)],
        out_hbm.at[c, pl.ds(s * ROWS_T, ROWS_T)],
    )


def _seg_body(src_hbm, dst_hbm, g_hbm, out_hbm, src_v, dst_v,
              buf0, buf1, buf2, acc_sh, gs0, gs1, gs2):
    c = lax.axis_index("c")
    s = lax.axis_index("s")
    wid = c * NS + s
    bufs = (buf0, buf1, buf2)
    gsem = (gs0, gs1, gs2)

    def zfill(r, carry):
        for j in range(D // 16):
            buf0[r, pl.ds(16 * j, 16)] = jnp.zeros((16,), jnp.float32)
        return carry

    lax.fori_loop(0, SPR, zfill, 0)
    for t in range(ROWS_T // SPR):
        pltpu.sync_copy(buf0, acc_sh.at[pl.ds(s * ROWS_T + t * SPR, SPR)])
    pltpu.sync_copy(
        buf0.at[pl.ds(0, REM_Z)],
        acc_sh.at[pl.ds(s * ROWS_T + (ROWS_T // SPR) * SPR, REM_Z)],
    )
    plsc.subcore_barrier()

    def gather(i, b):
        pltpu.async_copy(g_hbm.at[src_v.at[i]], bufs[b], gsem[b])

    def gather_wait(i, b):
        pltpu.make_async_copy(g_hbm.at[src_v.at[i]], bufs[b], gsem[b]).wait()

    for stage in range(NT2 // SCH):
        pltpu.sync_copy(src_hbm.at[wid, stage], src_v)
        pltpu.sync_copy(dst_hbm.at[wid, stage], dst_v)
        gather(0, 0)
        gather(1, 1)

        def tri(p, carry):
            for b in range(NBUF):
                i = p * NBUF + b
                nxt = i + 2

                gather_wait(i, b)

                @pl.when(nxt < SCH)
                def _():
                    gather(nxt, (b + 2) % NBUF)

                pltpu.sync_copy(bufs[b], acc_sh.at[dst_v.at[i]], add=True)
            return carry

        lax.fori_loop(0, SCH // NBUF, tri, 0)

    plsc.subcore_barrier()
    pltpu.sync_copy(
        acc_sh.at[pl.ds(s * ROWS_T, ROWS_T)],
        out_hbm.at[c, pl.ds(s * ROWS_T, ROWS_T)],
    )


def _lin_body(x_ref, w_ref, degp_ref, h_ref, g_ref):
    deg = degp_ref[0] + degp_ref[1] + 1.0
    h = jnp.dot(x_ref[...], w_ref[...], preferred_element_type=jnp.float32)
    h_ref[...] = h
    g_ref[...] = lax.rsqrt(deg) * h


def _comb_body(sp_ref, h_ref, degp_ref, b_ref, out_ref):
    deg = degp_ref[0] + degp_ref[1] + 1.0
    dis = lax.rsqrt(deg)
    s_sum = sp_ref[0] + sp_ref[1]
    out_ref[...] = dis * s_sum + (dis * dis) * h_ref[...] + b_ref[...]


_BLK = 1024
_GRID = NPAD // _BLK


@functools.lru_cache(maxsize=None)
def _sc_kernels():
    mesh = plsc.VectorSubcoreMesh(
        core_axis_name="c", subcore_axis_name="s", num_cores=NC, num_subcores=NS
    )
    deg_k = pl.kernel(
        _deg_body,
        out_type=jax.ShapeDtypeStruct((NC, NPAD), jnp.float32),
        mesh=mesh,
        scratch_types=[
            pltpu.VMEM((NT, IPR), jnp.int32),
            pltpu.VMEM((IPR,), jnp.float32),
            pltpu.VMEM((ROWS_T,), jnp.float32),
            pltpu.VMEM_SHARED((NPAD,), jnp.float32),
            pltpu.SemaphoreType.DMA,
        ],
    )
    seg_k = pl.kernel(
        _seg_body,
        out_type=jax.ShapeDtypeStruct((NC, NPAD, D), jnp.float32),
        mesh=mesh,
        scratch_types=(
            [
                pltpu.VMEM((SCH, SPR), jnp.int32),
                pltpu.VMEM((SCH, SPR), jnp.int32),
            ]
            + [pltpu.VMEM((SPR, D), jnp.float32)] * NBUF
            + [pltpu.VMEM_SHARED((NPAD, D), jnp.float32)]
            + [pltpu.SemaphoreType.DMA] * NBUF
        ),
    )
    return deg_k, seg_k


def kernel(input_x, edge_index, W, b):
    x = input_x.astype(jnp.float32)
    src = edge_index[0].astype(jnp.int32)
    dst = edge_index[1].astype(jnp.int32)

    n_extra1 = EPAD - N_EDGES
    fill1 = N_NODES + (jnp.arange(n_extra1, dtype=jnp.int32) % (NPAD - N_NODES))
    dst_p128 = jnp.concatenate([dst, fill1]).reshape(EPAD // IPR, IPR)

    n_extra2 = EPAD2 - N_EDGES
    fill2 = N_NODES + (jnp.arange(n_extra2, dtype=jnp.int32) % (NPAD - N_NODES))
    src_p = jnp.concatenate([src, fill2]).reshape(NW, NT2 // SCH, SCH, SPR)
    dst_p = jnp.concatenate([dst, fill2]).reshape(NW, NT2 // SCH, SCH, SPR)

    x_pad = jnp.pad(x, ((0, NPAD - N_NODES), (0, 0)))

    deg_kernel, seg_kernel = _sc_kernels()
    degp = deg_kernel(dst_p128)
    degp_col = degp.reshape(NC, NPAD, 1)

    h, g = pl.pallas_call(
        _lin_body,
        grid=(_GRID,),
        in_specs=[
            pl.BlockSpec((_BLK, D), lambda i: (i, 0)),
            pl.BlockSpec((D, D), lambda i: (0, 0)),
            pl.BlockSpec((NC, _BLK, 1), lambda i: (0, i, 0)),
        ],
        out_specs=[
            pl.BlockSpec((_BLK, D), lambda i: (i, 0)),
            pl.BlockSpec((_BLK, D), lambda i: (i, 0)),
        ],
        out_shape=[
            jax.ShapeDtypeStruct((NPAD, D), jnp.float32),
            jax.ShapeDtypeStruct((NPAD, D), jnp.float32),
        ],
    )(x_pad, W.astype(jnp.float32), degp_col)

    sp = seg_kernel(src_p, dst_p, g)

    out_pad = pl.pallas_call(
        _comb_body,
        grid=(_GRID,),
        in_specs=[
            pl.BlockSpec((NC, _BLK, D), lambda i: (0, i, 0)),
            pl.BlockSpec((_BLK, D), lambda i: (i, 0)),
            pl.BlockSpec((NC, _BLK, 1), lambda i: (0, i, 0)),
            pl.BlockSpec((1, D), lambda i: (0, 0)),
        ],
        out_specs=pl.BlockSpec((_BLK, D), lambda i: (i, 0)),
        out_shape=jax.ShapeDtypeStruct((NPAD, D), jnp.float32),
    )(sp, h, degp_col, b.astype(jnp.float32).reshape(1, D))

    return out_pad[:N_NODES]
